# baseline (speedup 1.0000x reference)
import jax
import jax.numpy as jnp
from jax.experimental import pallas as pl
from jax.experimental.pallas import tpu as pltpu
from jax.experimental.pallas import tpu_sc as plsc
import functools

S, B, H, V = 8, 16, 2048, 1000
SB = S * B
G = 3 * H

KC = 256
NK = H // KC
OC = 1024
NO = H // OC


_SC_WORKERS = 16
_ROWS_PER_W = SB // _SC_WORKERS


def _sc_gather_body(emb_hbm, idx_hbm, out_hbm, idx_v, rows_v, sem):
    nc = 2
    wid = jax.lax.axis_index("s") * nc + jax.lax.axis_index("c")

    @pl.when(wid < _SC_WORKERS)
    def _():
        base = wid * _ROWS_PER_W
        pltpu.sync_copy(idx_hbm.at[pl.ds(base, _ROWS_PER_W)], idx_v)
        pltpu.async_copy(emb_hbm.at[idx_v], rows_v, sem).wait()
        pltpu.sync_copy(rows_v, out_hbm.at[pl.ds(base, _ROWS_PER_W)])


def _sc_gather(emb, seq1d):
    mesh = plsc.VectorSubcoreMesh(core_axis_name="c", subcore_axis_name="s")
    k = functools.partial(
        pl.kernel,
        mesh=mesh,
        out_type=jax.ShapeDtypeStruct((SB, H), jnp.float32),
        scratch_types=[
            pltpu.VMEM((_ROWS_PER_W,), jnp.int32),
            pltpu.VMEM((_ROWS_PER_W, H), jnp.float32),
            pltpu.SemaphoreType.DMA,
        ],
    )(_sc_gather_body)
    return k(emb, seq1d)


def _sub_copies(hbm, buf, sem, row0, rows, nsub):
    sub = rows // nsub
    return [
        pltpu.make_async_copy(
            hbm.at[pl.ds(row0 + s * sub, sub), :],
            buf.at[pl.ds(s * sub, sub), :],
            sem,
        )
        for s in range(nsub)
    ]


def _issue(hbm, buf, sem, row0, rows, nsub):
    for c in _sub_copies(hbm, buf, sem, row0, rows, nsub):
        c.start()


def _wait(hbm, buf, sem, row0, rows, nsub):
    for c in _sub_copies(hbm, buf, sem, row0, rows, nsub):
        c.wait()


def _mega_body(inp_ref, binit_ref, bih_ref, bhh_ref, bout_ref,
               x_hbm, wi_hbm, wih_hbm, whh_hbm, wo_hbm,
               lp_ref, hs_ref,
               whh_bf, blk, wiblk, woblk, gi, x_buf, h_buf,
               sem_blk, sem_wi, sem_emb, sem_wo):
    f32 = jnp.float32

    def blk_src(j):
        return (wih_hbm, j * KC) if j < NK else (whh_hbm, (j - NK) * KC)

    def blk_issue(j):
        src, r0 = blk_src(j)
        _issue(src, blk.at[j % 2], sem_blk.at[j % 2], r0, KC, 8)

    def blk_wait(j):
        src, r0 = blk_src(j)
        _wait(src, blk.at[j % 2], sem_blk.at[j % 2], r0, KC, 8)

    _issue(wi_hbm, wiblk.at[0], sem_wi.at[0], 0, KC, 4)
    _issue(x_hbm, x_buf, sem_emb, 0, SB, 2)
    blk_issue(0)

    h_buf[...] = jnp.broadcast_to(binit_ref[...], (B, H))
    for k in range(NK):
        if k + 1 < NK:
            _issue(wi_hbm, wiblk.at[(k + 1) % 2], sem_wi.at[(k + 1) % 2],
                   (k + 1) * KC, KC, 4)
        _wait(wi_hbm, wiblk.at[k % 2], sem_wi.at[k % 2], k * KC, KC, 4)
        h_buf[...] += jnp.dot(inp_ref[:, k * KC:(k + 1) * KC],
                              wiblk[k % 2], preferred_element_type=f32)
    h_buf[...] = jnp.tanh(h_buf[...])

    _wait(x_hbm, x_buf, sem_emb, 0, SB, 2)

    gi[...] = jnp.broadcast_to(bih_ref[...], (SB, G))
    for j in range(2 * NK):
        if j + 1 < 2 * NK:
            blk_issue(j + 1)
        blk_wait(j)
        if j < NK:
            gi[...] += jnp.dot(x_buf[:, j * KC:(j + 1) * KC],
                               blk[j % 2], preferred_element_type=f32)
        else:
            k = j - NK
            whh_bf[pl.ds(k * KC, KC), :] = blk[j % 2].astype(jnp.bfloat16)

    _issue(wo_hbm, woblk.at[0], sem_wo.at[0], 0, OC, 4)
    _issue(wo_hbm, woblk.at[1], sem_wo.at[1], OC, OC, 4)

    def step(t, h):
        gh = jnp.dot(h.astype(jnp.bfloat16), whh_bf[...],
                     preferred_element_type=f32) + bhh_ref[...]
        gi_t = gi[pl.ds(t * B, B), :]
        r = jax.nn.sigmoid(gi_t[:, 0:H] + gh[:, 0:H])
        z = jax.nn.sigmoid(gi_t[:, H:2 * H] + gh[:, H:2 * H])
        n = jnp.tanh(gi_t[:, 2 * H:3 * H] + r * gh[:, 2 * H:3 * H])
        h_new = (1.0 - z) * n + z * h
        hs_ref[pl.ds(t * B, B), :] = h_new
        return h_new

    jax.lax.fori_loop(0, S, step, h_buf[...])

    lp_ref[...] = jnp.broadcast_to(bout_ref[...], (SB, V))
    for k in range(NO):
        _wait(wo_hbm, woblk.at[k % 2], sem_wo.at[k % 2], k * OC, OC, 4)
        lp_ref[...] += jnp.dot(hs_ref[:, k * OC:(k + 1) * OC],
                               woblk[k % 2], preferred_element_type=f32)
    logits = lp_ref[...]
    m = jnp.max(logits, axis=-1, keepdims=True)
    shifted = logits - m
    lse = jnp.log(jnp.sum(jnp.exp(shifted), axis=-1, keepdims=True))
    lp_ref[...] = shifted - lse


def kernel(seq_part, seq_length, input, emb, W_init, b_init, W_ih, W_hh, b_ih, b_hh, W_out, b_out):
    del seq_length
    seq1d = seq_part.reshape(SB)
    x_gathered = _sc_gather(emb, seq1d)

    vmem = pl.BlockSpec(memory_space=pltpu.VMEM)
    hbm = pl.BlockSpec(memory_space=pl.ANY)

    log_probs, hs = pl.pallas_call(
        _mega_body,
        in_specs=[vmem] * 5 + [hbm] * 5,
        out_specs=(vmem, vmem),
        out_shape=(
            jax.ShapeDtypeStruct((SB, V), jnp.float32),
            jax.ShapeDtypeStruct((SB, H), jnp.float32),
        ),
        scratch_shapes=[
            pltpu.VMEM((H, G), jnp.bfloat16),
            pltpu.VMEM((2, KC, G), jnp.float32),
            pltpu.VMEM((2, KC, H), jnp.float32),
            pltpu.VMEM((2, OC, V), jnp.float32),
            pltpu.VMEM((SB, G), jnp.float32),
            pltpu.VMEM((SB, H), jnp.float32),
            pltpu.VMEM((B, H), jnp.float32),
            pltpu.SemaphoreType.DMA((2,)),
            pltpu.SemaphoreType.DMA((2,)),
            pltpu.SemaphoreType.DMA,
            pltpu.SemaphoreType.DMA((2,)),
        ],
    )(input, b_init.reshape(1, H), b_ih.reshape(1, G),
      b_hh.reshape(1, G), b_out.reshape(1, V),
      x_gathered, W_init, W_ih, W_hh, W_out)

    hidden = hs[(S - 1) * B:].reshape(1, B, H)
    return log_probs.reshape(S, B, V), hidden
